# Initial kernel scaffold; baseline (speedup 1.0000x reference)
import jax
import jax.numpy as jnp
from jax.experimental import pallas as pl


def kernel(x, edge_index, W, b):
    raise NotImplementedError("write your pallas kernel here")


# speedup vs baseline: 6.1565x; 6.1565x over previous
import functools

import jax
import jax.numpy as jnp
from jax import lax
from jax.experimental import pallas as pl
from jax.experimental.pallas import tpu as pltpu
from jax.experimental.pallas import tpu_sc as plsc

N = 10000
E = 320000
D = 128
NC = 2
NS = 16
NW = NC * NS
CHUNK = 128
NCHUNKS = E // CHUNK
NPAD = 10240
RPT = NPAD // NS

_mesh = plsc.VectorSubcoreMesh(core_axis_name="c", subcore_axis_name="s")


@functools.partial(
    pl.kernel,
    out_type=(
        jax.ShapeDtypeStruct((NPAD,), jnp.float32),
        jax.ShapeDtypeStruct((NPAD,), jnp.float32),
    ),
    mesh=_mesh,
    scratch_types=[
        pltpu.VMEM((CHUNK,), jnp.int32),
        pltpu.VMEM((CHUNK,), jnp.float32),
        pltpu.VMEM((RPT,), jnp.float32),
        pltpu.VMEM_SHARED((NPAD,), jnp.float32),
    ],
)
def _deg_kernel(src_hbm, dst_hbm, deg0_hbm, deg1_hbm, idx_v, ones_v, buf_v,
                hist_sh):
    c = lax.axis_index("c")
    s = lax.axis_index("s")

    ones16 = jnp.ones((16,), jnp.float32)
    zeros16 = jnp.zeros((16,), jnp.float32)
    for k in range(CHUNK // 16):
        ones_v[pl.ds(k * 16, 16)] = ones16

    def zbody(i, _):
        buf_v[pl.ds(i * 16, 16)] = zeros16
        return 0

    lax.fori_loop(0, RPT // 16, zbody, 0)
    pltpu.sync_copy(buf_v, hist_sh.at[pl.ds(s * RPT, RPT)])
    plsc.subcore_barrier()

    def do_hist(edge_ref):
        def body(j, _):
            cid = s + j * NS
            base = pl.multiple_of(cid * CHUNK, CHUNK)
            pltpu.sync_copy(edge_ref.at[pl.ds(base, CHUNK)], idx_v)
            pltpu.sync_copy(ones_v, hist_sh.at[idx_v], add=True)
            return 0

        nch = (NCHUNKS - s + NS - 1) // NS
        lax.fori_loop(0, nch, body, 0)

    @pl.when(c == 0)
    def _():
        do_hist(src_hbm)

    @pl.when(c == 1)
    def _():
        do_hist(dst_hbm)

    plsc.subcore_barrier()
    off = pl.multiple_of(s * RPT, RPT)
    pltpu.sync_copy(hist_sh.at[pl.ds(off, RPT)], buf_v)

    @pl.when(c == 0)
    def _():
        pltpu.sync_copy(buf_v, deg0_hbm.at[pl.ds(off, RPT)])

    @pl.when(c == 1)
    def _():
        pltpu.sync_copy(buf_v, deg1_hbm.at[pl.ds(off, RPT)])


@functools.partial(
    pl.kernel,
    out_type=(
        jax.ShapeDtypeStruct((NPAD, D), jnp.float32),
        jax.ShapeDtypeStruct((NPAD, D), jnp.float32),
    ),
    mesh=_mesh,
    scratch_types=[
        pltpu.VMEM((CHUNK,), jnp.int32),
        pltpu.VMEM((CHUNK,), jnp.int32),
        pltpu.VMEM((CHUNK, D), jnp.float32),
        pltpu.VMEM((64, D), jnp.float32),
        pltpu.VMEM_SHARED((NPAD, D), jnp.float32),
        pltpu.SemaphoreType.DMA,
    ],
)
def _agg_kernel(feat_hbm, src_hbm, dst_hbm, out0_hbm, out1_hbm, si_v, di_v,
                rows_v, zero_v, acc_sh, sem):
    c = lax.axis_index("c")
    s = lax.axis_index("s")
    w = s * NC + c

    zeros16 = jnp.zeros((16,), jnp.float32)

    def zbody(i, _):
        for k in range(D // 16):
            zero_v[i, pl.ds(k * 16, 16)] = zeros16
        return 0

    lax.fori_loop(0, 64, zbody, 0)
    for t in range(RPT // 64):
        pltpu.sync_copy(zero_v, acc_sh.at[pl.ds(s * RPT + t * 64, 64)])
    plsc.subcore_barrier()

    def body(j, _):
        cid = w + j * NW
        base = pl.multiple_of(cid * CHUNK, CHUNK)
        pltpu.sync_copy(src_hbm.at[pl.ds(base, CHUNK)], si_v)
        pltpu.sync_copy(dst_hbm.at[pl.ds(base, CHUNK)], di_v)
        pltpu.async_copy(feat_hbm.at[si_v], rows_v, sem).wait()
        pltpu.sync_copy(rows_v, acc_sh.at[di_v], add=True)
        return 0

    nch = (NCHUNKS - w + NW - 1) // NW
    lax.fori_loop(0, nch, body, 0)
    plsc.subcore_barrier()

    off = pl.multiple_of(s * RPT, RPT)

    @pl.when(c == 0)
    def _():
        pltpu.sync_copy(acc_sh.at[pl.ds(off, RPT)], out0_hbm.at[pl.ds(off, RPT)])

    @pl.when(c == 1)
    def _():
        pltpu.sync_copy(acc_sh.at[pl.ds(off, RPT)], out1_hbm.at[pl.ds(off, RPT)])


def _feat_body(x_ref, deg_ref, o_ref):
    norm = lax.rsqrt(jnp.maximum(deg_ref[...], 1.0))
    o_ref[...] = x_ref[...] * norm


def _feat(x, deg_out):
    blk = 1000
    return pl.pallas_call(
        _feat_body,
        grid=(N // blk,),
        in_specs=[
            pl.BlockSpec((blk, D), lambda i: (i, 0)),
            pl.BlockSpec((blk, 1), lambda i: (i, 0)),
        ],
        out_specs=pl.BlockSpec((blk, D), lambda i: (i, 0)),
        out_shape=jax.ShapeDtypeStruct((N, D), jnp.float32),
    )(x, deg_out)


def _final_body(p0_ref, p1_ref, deg_ref, w_ref, b_ref, o_ref):
    norm = lax.rsqrt(jnp.maximum(deg_ref[...], 1.0))
    r = (p0_ref[...] + p1_ref[...]) * norm
    o_ref[...] = (
        jnp.dot(r, w_ref[...], preferred_element_type=jnp.float32) + b_ref[...]
    )


def _final(p0, p1, deg_in, W, b):
    blk = 1000
    return pl.pallas_call(
        _final_body,
        grid=(N // blk,),
        in_specs=[
            pl.BlockSpec((blk, D), lambda i: (i, 0)),
            pl.BlockSpec((blk, D), lambda i: (i, 0)),
            pl.BlockSpec((blk, 1), lambda i: (i, 0)),
            pl.BlockSpec((D, D), lambda i: (0, 0)),
            pl.BlockSpec((1, D), lambda i: (0, 0)),
        ],
        out_specs=pl.BlockSpec((blk, D), lambda i: (i, 0)),
        out_shape=jax.ShapeDtypeStruct((N, D), jnp.float32),
    )(p0, p1, deg_in, W, b)


@jax.jit
def kernel(x, edge_index, W, b):
    src = edge_index[0].astype(jnp.int32)
    dst = edge_index[1].astype(jnp.int32)
    deg_out, deg_in = _deg_kernel(src, dst)
    feat = _feat(x, deg_out[:N].reshape(N, 1))
    p0, p1 = _agg_kernel(feat, src, dst)
    out = _final(p0[:N], p1[:N], deg_in[:N].reshape(N, 1), W,
                 b.reshape(1, D))
    return out


# speedup vs baseline: 13.1824x; 2.1412x over previous
import functools

import jax
import jax.numpy as jnp
from jax import lax
from jax.experimental import pallas as pl
from jax.experimental.pallas import tpu as pltpu
from jax.experimental.pallas import tpu_sc as plsc

N = 10000
E = 320000
D = 128
NC = 2
NS = 16
NW = NC * NS
CHUNK = 128
NCHUNKS = E // CHUNK
NPAD = 10240
RPT = NPAD // NS
NCH1 = 157
NSTEP = 81
P = 3
NPAD3 = 10112
RPT3 = NPAD3 // NS

_mesh = plsc.VectorSubcoreMesh(core_axis_name="c", subcore_axis_name="s")


@functools.partial(
    pl.kernel,
    out_type=(
        jax.ShapeDtypeStruct((NPAD,), jnp.float32),
        jax.ShapeDtypeStruct((NPAD,), jnp.float32),
    ),
    mesh=_mesh,
    scratch_types=[
        pltpu.VMEM((NCH1, CHUNK), jnp.int32),
        pltpu.VMEM((CHUNK,), jnp.float32),
        pltpu.VMEM((RPT,), jnp.float32),
        pltpu.VMEM_SHARED((NPAD,), jnp.float32),
        pltpu.SemaphoreType.DMA,
    ],
)
def _deg_kernel(src_hbm, dst_hbm, deg0_hbm, deg1_hbm, idx_v, ones_v, buf_v,
                hist_sh, isem):
    c = lax.axis_index("c")
    s = lax.axis_index("s")

    ones16 = jnp.ones((16,), jnp.float32)
    zeros16 = jnp.zeros((16,), jnp.float32)
    for k in range(CHUNK // 16):
        ones_v[pl.ds(k * 16, 16)] = ones16

    def zbody(i, _):
        buf_v[pl.ds(i * 16, 16)] = zeros16
        return 0

    lax.fori_loop(0, RPT // 16, zbody, 0)
    pltpu.sync_copy(buf_v, hist_sh.at[pl.ds(s * RPT, RPT)])

    def preload(edge_ref):
        def pre(j, _):
            cid = jnp.minimum(s + j * NS, NCHUNKS - 1)
            base = pl.multiple_of(cid * CHUNK, CHUNK)
            pltpu.async_copy(edge_ref.at[pl.ds(base, CHUNK)], idx_v.at[j],
                             isem)
            return 0

        lax.fori_loop(0, NCH1, pre, 0)

        def drain(j, _):
            pltpu.make_async_copy(edge_ref.at[pl.ds(0, CHUNK)], idx_v.at[0],
                                  isem).wait()
            return 0

        lax.fori_loop(0, NCH1, drain, 0)

    @pl.when(c == 0)
    def _():
        preload(src_hbm)

    @pl.when(c == 1)
    def _():
        preload(dst_hbm)

    plsc.subcore_barrier()

    nch = (NCHUNKS - s + NS - 1) // NS

    def scat(j, _):
        @pl.when(j < nch)
        def _():
            pltpu.sync_copy(ones_v, hist_sh.at[idx_v.at[j]], add=True)

        return 0

    lax.fori_loop(0, NCH1, scat, 0)
    plsc.subcore_barrier()
    off = pl.multiple_of(s * RPT, RPT)
    pltpu.sync_copy(hist_sh.at[pl.ds(off, RPT)], buf_v)

    @pl.when(c == 0)
    def _():
        pltpu.sync_copy(buf_v, deg0_hbm.at[pl.ds(off, RPT)])

    @pl.when(c == 1)
    def _():
        pltpu.sync_copy(buf_v, deg1_hbm.at[pl.ds(off, RPT)])


@functools.partial(
    pl.kernel,
    out_type=(
        jax.ShapeDtypeStruct((NPAD3, D), jnp.float32),
        jax.ShapeDtypeStruct((NPAD3, D), jnp.float32),
    ),
    mesh=_mesh,
    scratch_types=(
        [pltpu.VMEM((CHUNK,), jnp.int32) for _ in range(P)]
        + [pltpu.VMEM((CHUNK,), jnp.int32) for _ in range(P)]
        + [pltpu.VMEM((CHUNK, D), jnp.float32) for _ in range(P)]
        + [pltpu.VMEM_SHARED((NPAD3, D), jnp.float32)]
        + [pltpu.SemaphoreType.DMA for _ in range(2 * P)]
    ),
)
def _agg_kernel(feat_hbm, src_hbm, dst_hbm, out0_hbm, out1_hbm,
                si0, si1, si2, di0, di1, di2, r0, r1, r2, acc_sh,
                i0, i1, i2, g0, g1, g2):
    c = lax.axis_index("c")
    s = lax.axis_index("s")
    w = s * NC + c
    si = (si0, si1, si2)
    di = (di0, di1, di2)
    rows = (r0, r1, r2)
    isems = (i0, i1, i2)
    gsems = (g0, g1, g2)

    zeros16 = jnp.zeros((16,), jnp.float32)

    def zbody(i, _):
        for k in range(D // 16):
            r0[i, pl.ds(k * 16, 16)] = zeros16
        return 0

    lax.fori_loop(0, CHUNK, zbody, 0)
    zbase = s * RPT3
    for t in range(4):
        pltpu.async_copy(r0, acc_sh.at[pl.ds(zbase + t * CHUNK, CHUNK)], i0)
    pltpu.async_copy(r0.at[pl.ds(0, RPT3 - 4 * CHUNK)],
                     acc_sh.at[pl.ds(zbase + 4 * CHUNK, RPT3 - 4 * CHUNK)], i0)
    for t in range(4):
        pltpu.make_async_copy(
            r0, acc_sh.at[pl.ds(zbase + t * CHUNK, CHUNK)], i0).wait()
    pltpu.make_async_copy(
        r0.at[pl.ds(0, RPT3 - 4 * CHUNK)],
        acc_sh.at[pl.ds(zbase + 4 * CHUNK, RPT3 - 4 * CHUNK)], i0).wait()
    plsc.subcore_barrier()

    nch = (NCHUNKS - w + NW - 1) // NW

    def fire_idx(j, p):
        cid = jnp.minimum(w + j * NW, NCHUNKS - 1)
        base = pl.multiple_of(cid * CHUNK, CHUNK)
        pltpu.async_copy(src_hbm.at[pl.ds(base, CHUNK)], si[p], isems[p])
        pltpu.async_copy(dst_hbm.at[pl.ds(base, CHUNK)], di[p], isems[p])

    def wait_idx(p):
        pltpu.make_async_copy(src_hbm.at[pl.ds(0, CHUNK)], si[p],
                              isems[p]).wait()
        pltpu.make_async_copy(src_hbm.at[pl.ds(0, CHUNK)], di[p],
                              isems[p]).wait()

    def fire_g(p):
        pltpu.async_copy(feat_hbm.at[si[p]], rows[p], gsems[p])

    def wait_g(p):
        pltpu.make_async_copy(feat_hbm.at[si[p]], rows[p], gsems[p]).wait()

    for p in range(P):
        fire_idx(p, p)
    wait_idx(0)
    fire_g(0)

    def tri(t, _):
        for p in range(P):
            j = t * P + p
            p1 = (p + 1) % P

            @pl.when(j + 1 < NSTEP)
            def _():
                wait_idx(p1)
                fire_g(p1)

            wait_g(p)

            @pl.when(j < nch)
            def _():
                pltpu.sync_copy(rows[p], acc_sh.at[di[p]], add=True)

            @pl.when(j + P < NSTEP)
            def _():
                fire_idx(j + P, p)

        return 0

    lax.fori_loop(0, NSTEP // P, tri, 0)
    plsc.subcore_barrier()

    off3 = pl.multiple_of(s * RPT3, RPT3)

    @pl.when(c == 0)
    def _():
        pltpu.sync_copy(acc_sh.at[pl.ds(off3, RPT3)],
                        out0_hbm.at[pl.ds(off3, RPT3)])

    @pl.when(c == 1)
    def _():
        pltpu.sync_copy(acc_sh.at[pl.ds(off3, RPT3)],
                        out1_hbm.at[pl.ds(off3, RPT3)])


def _feat_body(x_ref, deg_ref, o_ref):
    norm = lax.rsqrt(jnp.maximum(deg_ref[...], 1.0))
    o_ref[...] = x_ref[...] * norm


def _feat(x, deg_out):
    blk = 1000
    return pl.pallas_call(
        _feat_body,
        grid=(N // blk,),
        in_specs=[
            pl.BlockSpec((blk, D), lambda i: (i, 0)),
            pl.BlockSpec((blk, 1), lambda i: (i, 0)),
        ],
        out_specs=pl.BlockSpec((blk, D), lambda i: (i, 0)),
        out_shape=jax.ShapeDtypeStruct((N, D), jnp.float32),
    )(x, deg_out)


def _final_body(p0_ref, p1_ref, deg_ref, w_ref, b_ref, o_ref):
    norm = lax.rsqrt(jnp.maximum(deg_ref[...], 1.0))
    r = (p0_ref[...] + p1_ref[...]) * norm
    o_ref[...] = (
        jnp.dot(r, w_ref[...], preferred_element_type=jnp.float32) + b_ref[...]
    )


def _final(p0, p1, deg_in, W, b):
    blk = 1000
    return pl.pallas_call(
        _final_body,
        grid=(N // blk,),
        in_specs=[
            pl.BlockSpec((blk, D), lambda i: (i, 0)),
            pl.BlockSpec((blk, D), lambda i: (i, 0)),
            pl.BlockSpec((blk, 1), lambda i: (i, 0)),
            pl.BlockSpec((D, D), lambda i: (0, 0)),
            pl.BlockSpec((1, D), lambda i: (0, 0)),
        ],
        out_specs=pl.BlockSpec((blk, D), lambda i: (i, 0)),
        out_shape=jax.ShapeDtypeStruct((N, D), jnp.float32),
    )(p0, p1, deg_in, W, b)


@jax.jit
def kernel(x, edge_index, W, b):
    src = edge_index[0].astype(jnp.int32)
    dst = edge_index[1].astype(jnp.int32)
    deg_out, deg_in = _deg_kernel(src, dst)
    feat = _feat(x, deg_out[:N].reshape(N, 1))
    p0, p1 = _agg_kernel(feat, src, dst)
    out = _final(p0, p1, deg_in[:N].reshape(N, 1), W, b.reshape(1, D))
    return out
